# baseline (speedup 1.0000x reference)
import jax
import jax.numpy as jnp
from jax.experimental import pallas as pl
from jax.experimental.pallas import tpu as pltpu

_LANES = 128
_ROWS = 136
_FROWS = 5 * _ROWS
_M = 32


def _shift_left(x, k):
    rows, lanes = x.shape
    return jnp.concatenate(
        [x[:, k:], jnp.zeros((rows, k), x.dtype)], axis=1)


def _loss_body(c_ref, cnt_ref, rp_ref, rt_ref, t_ref, thres_ref, out_ref):
    b = pl.program_id(0)
    nb = pl.num_programs(0)

    @pl.when(b == 0)
    def _():
        out_ref[...] = jnp.zeros((1, 1), jnp.float32)

    zero = jnp.float32(0.0)
    f32 = jnp.float32
    thres = thres_ref[0]
    c1 = f32(1.0) + thres

    tx1 = [t_ref[b, m, 0] for m in range(_M)]
    ty1 = [t_ref[b, m, 1] for m in range(_M)]
    tx2 = [t_ref[b, m, 2] for m in range(_M)]
    ty2 = [t_ref[b, m, 3] for m in range(_M)]
    atm = [thres * (jnp.maximum(tx2[m] - tx1[m], zero)
                    * jnp.maximum(ty2[m] - ty1[m], zero)) for m in range(_M)]

    pos_chunks = []
    for r0, rows in ((0, 72), (72, 64)):
        sl = pl.ds(r0, rows)
        px1 = c_ref[0, sl, :]
        py1 = c_ref[1, sl, :]
        px2 = c_ref[2, sl, :]
        py2 = c_ref[3, sl, :]
        apt = thres * (jnp.maximum(px2 - px1, zero)
                       * jnp.maximum(py2 - py1, zero))

        hit = jnp.zeros((rows, _LANES), jnp.bool_)
        for m in range(_M):
            w = jnp.maximum(
                jnp.minimum(px2, tx2[m]) - jnp.maximum(px1, tx1[m]), zero)
            h = jnp.minimum(py2, ty2[m]) - jnp.maximum(py1, ty1[m])
            inter = w * h
            hit = jnp.logical_or(hit, inter * c1 > apt + atm[m])

        pos_chunks.append(
            jnp.where(jnp.logical_and(hit, cnt_ref[sl, :] > zero),
                      f32(1.0), zero))
    pos = jnp.concatenate(pos_chunks, axis=0)

    d = rp_ref[...] - rt_ref[...]
    ad = jnp.abs(d)
    sl1 = jnp.where(ad < f32(1.0), f32(0.5) * d * d, ad - f32(0.5))

    s = sl1
    for k in range(1, 5):
        s = s + _shift_left(sl1, k)

    lane = jax.lax.broadcasted_iota(jnp.int32, (_ROWS, _LANES), 1)
    rowsum = jnp.zeros((_ROWS, _LANES), jnp.float32)
    for j in range(5):
        sj = s[:, 128 * j:128 * (j + 1)]
        idx = 5 * lane - 128 * j
        valid = jnp.logical_and(idx >= 0, idx < _LANES)
        idxc = jnp.clip(idx, 0, _LANES - 1)
        g = jnp.take_along_axis(sj, idxc, axis=1)
        rowsum = rowsum + jnp.where(valid, g, zero)

    npos_acc = jnp.zeros((8, _LANES), jnp.float32)
    loss_acc = jnp.zeros((8, _LANES), jnp.float32)
    contrib = rowsum * pos
    for v in range(_ROWS // 8):
        npos_acc = npos_acc + pos[8 * v:8 * v + 8, :]
        loss_acc = loss_acc + contrib[8 * v:8 * v + 8, :]

    npos = jnp.sum(npos_acc)
    lsum = jnp.sum(loss_acc)
    img = jnp.where(npos > zero, lsum / npos, zero)
    out_ref[...] = out_ref[...] + (img / f32(nb)).reshape(1, 1)


def kernel(P_bbx, cls_logits, reg_preds, T_boxes, cnt_p57, reg_p57, cnt_p2,
           reg_p2, iou_thres):
    del cls_logits
    B, N, _ = P_bbx.shape
    npad = _ROWS * _LANES

    coords = jnp.pad(P_bbx, ((0, 0), (0, npad - N), (0, 0))).transpose(
        0, 2, 1).reshape(B, 4, _ROWS, _LANES)
    cnt = jnp.pad(
        jnp.concatenate([cnt_p2.reshape(B, -1), cnt_p57.reshape(B, -1)],
                        axis=1),
        ((0, 0), (0, npad - N))).reshape(B, _ROWS, _LANES)
    row_pads = ((0, 0), (0, npad - N), (0, 0))
    rp = jnp.pad(reg_preds, row_pads).reshape(B, _ROWS, 640)
    rt = jnp.pad(
        jnp.concatenate([reg_p2.reshape(B, -1, 5), reg_p57], axis=1),
        row_pads).reshape(B, _ROWS, 640)
    thres = jnp.reshape(iou_thres, (1,)).astype(jnp.float32)

    out = pl.pallas_call(
        _loss_body,
        grid=(B,),
        in_specs=[
            pl.BlockSpec((None, 4, _ROWS, _LANES), lambda b: (b, 0, 0, 0)),
            pl.BlockSpec((None, _ROWS, _LANES), lambda b: (b, 0, 0)),
            pl.BlockSpec((None, _ROWS, 640), lambda b: (b, 0, 0)),
            pl.BlockSpec((None, _ROWS, 640), lambda b: (b, 0, 0)),
            pl.BlockSpec(memory_space=pltpu.SMEM),
            pl.BlockSpec(memory_space=pltpu.SMEM),
        ],
        out_specs=pl.BlockSpec((1, 1), lambda b: (0, 0)),
        out_shape=jax.ShapeDtypeStruct((1, 1), jnp.float32),
    )(coords, cnt, rp, rt, T_boxes, thres)
    return out


# speedup vs baseline: 9.8658x; 9.8658x over previous
import jax
import jax.numpy as jnp
from jax.experimental import pallas as pl
from jax.experimental.pallas import tpu as pltpu

_LANES = 128
_ROWS = 136
_M = 32


def _loss_body(c_ref, rp_ref, rt_ref, cnt_ref, t_ref, thres_ref, out_ref):
    b = pl.program_id(0)
    nb = pl.num_programs(0)

    @pl.when(b == 0)
    def _():
        out_ref[...] = jnp.zeros((1, 1), jnp.float32)

    zero = jnp.float32(0.0)
    thres = thres_ref[0]
    c1 = jnp.float32(1.0) + thres

    tx1 = [t_ref[b, m, 0] for m in range(_M)]
    ty1 = [t_ref[b, m, 1] for m in range(_M)]
    tx2 = [t_ref[b, m, 2] for m in range(_M)]
    ty2 = [t_ref[b, m, 3] for m in range(_M)]
    atm = [thres * (jnp.maximum(tx2[m] - tx1[m], zero)
                    * jnp.maximum(ty2[m] - ty1[m], zero)) for m in range(_M)]

    npos_acc = jnp.zeros((8, _LANES), jnp.float32)
    loss_acc = jnp.zeros((8, _LANES), jnp.float32)

    for r0, rows in ((0, 72), (72, 64)):
        sl = pl.ds(r0, rows)
        px1 = c_ref[0, sl, :]
        py1 = c_ref[1, sl, :]
        px2 = c_ref[2, sl, :]
        py2 = c_ref[3, sl, :]
        apt = thres * (jnp.maximum(px2 - px1, zero)
                       * jnp.maximum(py2 - py1, zero))

        hit = jnp.zeros((rows, _LANES), jnp.bool_)
        for m in range(_M):
            w = jnp.maximum(
                jnp.minimum(px2, tx2[m]) - jnp.maximum(px1, tx1[m]), zero)
            h = jnp.minimum(py2, ty2[m]) - jnp.maximum(py1, ty1[m])
            inter = w * h
            hit = jnp.logical_or(hit, inter * c1 > apt + atm[m])

        pos = jnp.where(jnp.logical_and(hit, cnt_ref[sl, :] > zero),
                        jnp.float32(1.0), zero)

        rowsum = jnp.zeros((rows, _LANES), jnp.float32)
        for k in range(5):
            d = rp_ref[k, sl, :] - rt_ref[k, sl, :]
            ad = jnp.abs(d)
            rowsum = rowsum + jnp.where(
                ad < jnp.float32(1.0),
                jnp.float32(0.5) * d * d, ad - jnp.float32(0.5))

        for v in range(rows // 8):
            npos_acc = npos_acc + pos[8 * v:8 * v + 8, :]
            loss_acc = loss_acc + (rowsum * pos)[8 * v:8 * v + 8, :]

    npos = jnp.sum(npos_acc)
    lsum = jnp.sum(loss_acc)
    img = jnp.where(npos > zero, lsum / npos, zero)
    out_ref[...] = out_ref[...] + (img / jnp.float32(nb)).reshape(1, 1)


def kernel(P_bbx, cls_logits, reg_preds, T_boxes, cnt_p57, reg_p57, cnt_p2,
           reg_p2, iou_thres):
    del cls_logits
    B, N, _ = P_bbx.shape
    npad = _ROWS * _LANES

    pads = ((0, 0), (0, npad - N), (0, 0))
    coords = jnp.pad(P_bbx, pads).transpose(0, 2, 1).reshape(
        B, 4, _ROWS, _LANES)
    rp = jnp.pad(reg_preds, pads).transpose(0, 2, 1).reshape(
        B, 5, _ROWS, _LANES)
    reg_t = jnp.concatenate([reg_p2.reshape(B, -1, 5), reg_p57], axis=1)
    rt = jnp.pad(reg_t, pads).transpose(0, 2, 1).reshape(
        B, 5, _ROWS, _LANES)
    cnt = jnp.pad(
        jnp.concatenate([cnt_p2.reshape(B, -1), cnt_p57.reshape(B, -1)],
                        axis=1),
        ((0, 0), (0, npad - N))).reshape(B, _ROWS, _LANES)
    thres = jnp.reshape(iou_thres, (1,)).astype(jnp.float32)

    out = pl.pallas_call(
        _loss_body,
        grid=(B,),
        in_specs=[
            pl.BlockSpec((None, 4, _ROWS, _LANES), lambda b: (b, 0, 0, 0)),
            pl.BlockSpec((None, 5, _ROWS, _LANES), lambda b: (b, 0, 0, 0)),
            pl.BlockSpec((None, 5, _ROWS, _LANES), lambda b: (b, 0, 0, 0)),
            pl.BlockSpec((None, _ROWS, _LANES), lambda b: (b, 0, 0)),
            pl.BlockSpec(memory_space=pltpu.SMEM),
            pl.BlockSpec(memory_space=pltpu.SMEM),
        ],
        out_specs=pl.BlockSpec((1, 1), lambda b: (0, 0)),
        out_shape=jax.ShapeDtypeStruct((1, 1), jnp.float32),
    )(coords, rp, rt, cnt, T_boxes, thres)
    return out


# speedup vs baseline: 10.2488x; 1.0388x over previous
import jax
import jax.numpy as jnp
from jax.experimental import pallas as pl
from jax.experimental.pallas import tpu as pltpu

_LANES = 128
_ROWS = 136
_M = 32


def _loss_body(c_ref, r_ref, cnt_ref, t_ref, thres_ref, out_ref):
    b = pl.program_id(0)
    nb = pl.num_programs(0)

    @pl.when(b == 0)
    def _():
        out_ref[...] = jnp.zeros((1, 1), jnp.float32)

    zero = jnp.float32(0.0)
    thres = thres_ref[0]
    c1 = jnp.float32(1.0) + thres

    tx1 = [t_ref[b, m, 0] for m in range(_M)]
    ty1 = [t_ref[b, m, 1] for m in range(_M)]
    tx2 = [t_ref[b, m, 2] for m in range(_M)]
    ty2 = [t_ref[b, m, 3] for m in range(_M)]
    atm = [thres * (jnp.maximum(tx2[m] - tx1[m], zero)
                    * jnp.maximum(ty2[m] - ty1[m], zero)) for m in range(_M)]

    npos_acc = jnp.zeros((8, _LANES), jnp.float32)
    loss_acc = jnp.zeros((8, _LANES), jnp.float32)

    for r0, rows in ((0, 72), (72, 64)):
        sl = pl.ds(r0, rows)
        px1 = c_ref[0, sl, :]
        py1 = c_ref[1, sl, :]
        px2 = c_ref[2, sl, :]
        py2 = c_ref[3, sl, :]
        apt = thres * (jnp.maximum(px2 - px1, zero)
                       * jnp.maximum(py2 - py1, zero))

        hit = jnp.zeros((rows, _LANES), jnp.bool_)
        for m in range(_M):
            w = jnp.maximum(
                jnp.minimum(px2, tx2[m]) - jnp.maximum(px1, tx1[m]), zero)
            h = jnp.minimum(py2, ty2[m]) - jnp.maximum(py1, ty1[m])
            inter = w * h
            hit = jnp.logical_or(hit, inter * c1 > apt + atm[m])

        pos = jnp.where(jnp.logical_and(hit, cnt_ref[sl, :] > zero),
                        jnp.float32(1.0), zero)

        rowsum = jnp.zeros((rows, _LANES), jnp.float32)
        for k in range(5):
            d = r_ref[k, sl, :] - r_ref[5 + k, sl, :]
            ad = jnp.abs(d)
            rowsum = rowsum + jnp.where(
                ad < jnp.float32(1.0),
                jnp.float32(0.5) * d * d, ad - jnp.float32(0.5))

        for v in range(rows // 8):
            npos_acc = npos_acc + pos[8 * v:8 * v + 8, :]
            loss_acc = loss_acc + (rowsum * pos)[8 * v:8 * v + 8, :]

    npos = jnp.sum(npos_acc)
    lsum = jnp.sum(loss_acc)
    img = jnp.where(npos > zero, lsum / npos, zero)
    out_ref[...] = out_ref[...] + (img / jnp.float32(nb)).reshape(1, 1)


def kernel(P_bbx, cls_logits, reg_preds, T_boxes, cnt_p57, reg_p57, cnt_p2,
           reg_p2, iou_thres):
    del cls_logits
    B, N, _ = P_bbx.shape
    npad = _ROWS * _LANES

    pads = ((0, 0), (0, npad - N), (0, 0))
    coords = jnp.pad(P_bbx, pads).transpose(0, 2, 1).reshape(
        B, 4, _ROWS, _LANES)
    reg_t = jnp.concatenate([reg_p2.reshape(B, -1, 5), reg_p57], axis=1)
    regs = jnp.pad(jnp.concatenate([reg_preds, reg_t], axis=2),
                   pads).transpose(0, 2, 1).reshape(B, 10, _ROWS, _LANES)
    cnt = jnp.pad(
        jnp.concatenate([cnt_p2.reshape(B, -1), cnt_p57.reshape(B, -1)],
                        axis=1),
        ((0, 0), (0, npad - N))).reshape(B, _ROWS, _LANES)
    thres = jnp.reshape(iou_thres, (1,)).astype(jnp.float32)

    out = pl.pallas_call(
        _loss_body,
        grid=(B,),
        in_specs=[
            pl.BlockSpec((None, 4, _ROWS, _LANES), lambda b: (b, 0, 0, 0)),
            pl.BlockSpec((None, 10, _ROWS, _LANES), lambda b: (b, 0, 0, 0)),
            pl.BlockSpec((None, _ROWS, _LANES), lambda b: (b, 0, 0)),
            pl.BlockSpec(memory_space=pltpu.SMEM),
            pl.BlockSpec(memory_space=pltpu.SMEM),
        ],
        out_specs=pl.BlockSpec((1, 1), lambda b: (0, 0)),
        out_shape=jax.ShapeDtypeStruct((1, 1), jnp.float32),
    )(coords, regs, cnt, T_boxes, thres)
    return out
